# baseline (speedup 1.0000x reference)
import functools

import numpy as np
import jax
import jax.numpy as jnp
from jax import lax
from jax.experimental import pallas as pl
from jax.experimental.pallas import tpu as pltpu
from jax.experimental.pallas import tpu_sc as plsc

_N, _L, _C = 128, 8192, 20
_ROWS = _N * _L
_NW = 32
_ROWS_PER_W = _ROWS // _NW
_CHUNK = 2048
_NCHUNK = _ROWS_PER_W // _CHUNK
_GROUPS = _CHUNK // 16


def _smooth_table() -> np.ndarray:
    vals = np.array([0.0024, 0.0763, 0.8426, 0.0763, 0.0024], dtype=np.float32)
    tab = np.zeros((20, 20), dtype=np.float32)
    for i in range(20):
        for k in range(5):
            p = i + k - 2
            if 0 <= p < 20:
                tab[i, p] += vals[k]
    return tab / tab.sum(axis=1, keepdims=True)


_TABLE = _smooth_table().reshape(-1)


@functools.partial(
    pl.kernel,
    out_type=jax.ShapeDtypeStruct((_NW, 16), jnp.float32),
    mesh=plsc.VectorSubcoreMesh(core_axis_name="c", subcore_axis_name="s"),
    compiler_params=pltpu.CompilerParams(needs_layout_passes=False),
    scratch_types=[
        pltpu.VMEM((_CHUNK * _C,), jnp.float32),
        pltpu.VMEM((_CHUNK * _C,), jnp.float32),
        pltpu.VMEM((_CHUNK,), jnp.float32),
        pltpu.VMEM((_CHUNK,), jnp.float32),
        pltpu.VMEM((400,), jnp.float32),
        pltpu.VMEM((16,), jnp.float32),
        pltpu.SemaphoreType.DMA,
        pltpu.SemaphoreType.DMA,
        pltpu.SemaphoreType.DMA,
        pltpu.SemaphoreType.DMA,
    ],
)
def _edm_sc(x_hbm, t_hbm, tab_hbm, out_hbm,
            xba, xbb, tba, tbb, tab_v, acc_v,
            sxa, sxb, sta, stb):
    wid = lax.axis_index("s") * 2 + lax.axis_index("c")
    base_row = wid * _ROWS_PER_W
    lane20 = lax.broadcasted_iota(jnp.int32, (16,), 0) * 20

    def start(ch, xbuf, tbuf, sx, st):
        row0 = base_row + ch * _CHUNK
        pltpu.make_async_copy(
            x_hbm.at[pl.ds(row0 * _C, _CHUNK * _C)], xbuf, sx).start()
        pltpu.make_async_copy(
            t_hbm.at[pl.ds(row0, _CHUNK)], tbuf, st).start()

    def wait(xbuf, tbuf, sx, st):
        pltpu.make_async_copy(
            x_hbm.at[pl.ds(0, _CHUNK * _C)], xbuf, sx).wait()
        pltpu.make_async_copy(
            t_hbm.at[pl.ds(0, _CHUNK)], tbuf, st).wait()

    def compute(xbuf, tbuf):
        @pl.loop(0, _GROUPS)
        def _(g):
            t = tbuf[pl.ds(g * 16, 16)]
            f = t * 100.0 - 100.0
            xi = f.astype(jnp.int32)
            xi = jnp.minimum(jnp.maximum(xi, 0), 399)
            bin20 = jnp.right_shift(xi * 3277, 16) * 20
            xbase = g * 320 + lane20
            run = jnp.zeros((16,), jnp.float32)
            ssq = jnp.zeros((16,), jnp.float32)
            for c in range(_C):
                xc = plsc.load_gather(xbuf, [xbase + c])
                sc = plsc.load_gather(tab_v, [bin20 + c])
                run = run + (xc - sc)
                ssq = ssq + run * run
            y = ssq * (1.0 / 20.0) + 1e-6
            r = plsc.bitcast(
                jnp.int32(0x5F3759DF)
                - jnp.right_shift(plsc.bitcast(y, jnp.int32), 1),
                jnp.float32)
            for _ in range(3):
                r = r * (1.5 - 0.5 * y * r * r)
            acc_v[...] = acc_v[...] + y * r

    pltpu.sync_copy(tab_hbm, tab_v)
    acc_v[...] = jnp.zeros((16,), jnp.float32)
    start(0, xba, tba, sxa, sta)

    @pl.loop(0, _NCHUNK, step=2)
    def _(ch):
        start(ch + 1, xbb, tbb, sxb, stb)
        wait(xba, tba, sxa, sta)
        compute(xba, tba)

        @pl.when(ch + 2 < _NCHUNK)
        def _():
            start(ch + 2, xba, tba, sxa, sta)

        wait(xbb, tbb, sxb, stb)
        compute(xbb, tbb)

    pltpu.sync_copy(acc_v, out_hbm.at[wid])


def kernel(input, target):
    x = input.reshape(_ROWS * _C)
    t = target.reshape(_ROWS)
    parts = _edm_sc(x, t, jnp.asarray(_TABLE))
    return jnp.sum(parts) * (1.0 / _ROWS)


# speedup vs baseline: 3.2664x; 3.2664x over previous
import functools

import numpy as np
import jax
import jax.numpy as jnp
from jax import lax
from jax.experimental import pallas as pl
from jax.experimental.pallas import tpu as pltpu
from jax.experimental.pallas import tpu_sc as plsc

_N, _L, _C = 128, 8192, 20
_ROWS = _N * _L
_NW = 32
_ROWS_PER_W = _ROWS // _NW
_CHUNK = 2048
_NCHUNK = _ROWS_PER_W // _CHUNK
_GROUPS = _CHUNK // 16


def _smooth_table() -> np.ndarray:
    vals = np.array([0.0024, 0.0763, 0.8426, 0.0763, 0.0024], dtype=np.float32)
    tab = np.zeros((20, 21), dtype=np.float32)
    for i in range(20):
        for k in range(5):
            p = i + k - 2
            if 0 <= p < 20:
                tab[i, p] += vals[k]
    return tab / tab.sum(axis=1, keepdims=True)


_TABLE = _smooth_table().reshape(-1)


@functools.partial(
    pl.kernel,
    out_type=jax.ShapeDtypeStruct((_NW * 16,), jnp.float32),
    mesh=plsc.VectorSubcoreMesh(core_axis_name="c", subcore_axis_name="s"),
    compiler_params=pltpu.CompilerParams(needs_layout_passes=False),
    scratch_types=[
        pltpu.VMEM((_C * _CHUNK // 128, 128), jnp.float32),
        pltpu.VMEM((_C * _CHUNK // 128, 128), jnp.float32),
        pltpu.VMEM((_CHUNK // 128, 128), jnp.float32),
        pltpu.VMEM((_CHUNK // 128, 128), jnp.float32),
        pltpu.VMEM((420,), jnp.float32),
        pltpu.VMEM((16,), jnp.float32),
        pltpu.SemaphoreType.DMA,
        pltpu.SemaphoreType.DMA,
        pltpu.SemaphoreType.DMA,
        pltpu.SemaphoreType.DMA,
    ],
)
def _edm_sc(x_hbm, t_hbm, tab_hbm, out_hbm,
            xba, xbb, tba, tbb, tab_v, acc_v,
            sxa, sxb, sta, stb):
    wid = lax.axis_index("s") * 2 + lax.axis_index("c")
    base_row = wid * _ROWS_PER_W

    def start(ch, xbuf, tbuf, sx, st):
        row0 = base_row + ch * _CHUNK
        for c in range(_C):
            off = pl.multiple_of((c * _ROWS + row0) // 128, 16)
            pltpu.make_async_copy(
                x_hbm.at[pl.ds(off, _CHUNK // 128), :],
                xbuf.at[pl.ds(c * (_CHUNK // 128), _CHUNK // 128), :],
                sx).start()
        pltpu.make_async_copy(
            t_hbm.at[pl.ds(pl.multiple_of(row0 // 128, 16), _CHUNK // 128), :],
            tbuf, st).start()

    def wait(xbuf, tbuf, sx, st):
        pltpu.make_async_copy(
            x_hbm.at[pl.ds(0, _C * _CHUNK // 128), :], xbuf, sx).wait()
        pltpu.make_async_copy(
            t_hbm.at[pl.ds(0, _CHUNK // 128), :], tbuf, st).wait()

    def compute(xbuf, tbuf):
        @pl.loop(0, _GROUPS)
        def _(g):
            grow = g // 8
            gcol = (g % 8) * 16
            t = tbuf[grow, pl.ds(gcol, 16)]
            f = t * 100.0 - 100.0
            xi = f.astype(jnp.int32)
            xi = jnp.minimum(jnp.maximum(xi, 0), 399)
            bin21 = jnp.right_shift(xi * 3277, 16) * 21
            run = jnp.zeros((16,), jnp.float32)
            ssq = jnp.zeros((16,), jnp.float32)
            for c in range(_C):
                xc = xbuf[c * (_CHUNK // 128) + grow, pl.ds(gcol, 16)]
                sc = plsc.load_gather(tab_v, [bin21 + c])
                run = run + (xc - sc)
                ssq = ssq + run * run
            y = ssq * (1.0 / 20.0) + 1e-6
            r = plsc.bitcast(
                jnp.int32(0x5F3759DF)
                - jnp.right_shift(plsc.bitcast(y, jnp.int32), 1),
                jnp.float32)
            for _ in range(3):
                r = r * (1.5 - 0.5 * y * r * r)
            acc_v[...] = acc_v[...] + y * r

    pltpu.sync_copy(tab_hbm, tab_v)
    acc_v[...] = jnp.zeros((16,), jnp.float32)
    start(0, xba, tba, sxa, sta)

    @pl.loop(0, _NCHUNK, step=2)
    def _(ch):
        start(ch + 1, xbb, tbb, sxb, stb)
        wait(xba, tba, sxa, sta)
        compute(xba, tba)

        @pl.when(ch + 2 < _NCHUNK)
        def _():
            start(ch + 2, xba, tba, sxa, sta)

        wait(xbb, tbb, sxb, stb)
        compute(xbb, tbb)

    pltpu.sync_copy(acc_v, out_hbm.at[pl.ds(wid * 16, 16)])


def kernel(input, target):
    x = input.transpose(2, 0, 1).reshape(_C * _ROWS // 128, 128)
    t = target.reshape(_ROWS // 128, 128)
    parts = _edm_sc(x, t, jnp.asarray(_TABLE))
    return jnp.sum(parts) * (1.0 / _ROWS)
